# baseline (speedup 1.0000x reference)
import functools

import jax
import jax.numpy as jnp
from jax import lax
from jax.experimental import pallas as pl
from jax.experimental.pallas import tpu as pltpu
from jax.experimental.pallas import tpu_sc as plsc

N = 10000
N_PAD = 10240
DIN = 128
DOUT = 64
E = 320000
CHUNK = 128


def _sc_dims():
    info = plsc.get_sparse_core_info()
    return info.num_cores, info.num_subcores


GRP = 8
NBUF = 2


@functools.lru_cache(maxsize=None)
def _make_sc_agg(with_gather: bool, nc: int, ns: int, cpw: int):
    rpt = N_PAD // ns
    nrc = rpt // CHUNK
    gpw = cpw // GRP

    mesh = plsc.VectorSubcoreMesh(core_axis_name="c", subcore_axis_name="s")
    out_type = jax.ShapeDtypeStruct((nc * N_PAD, DIN), jnp.float32)

    scratch = (
        [pltpu.VMEM_SHARED((N_PAD, DIN), jnp.float32)]
        + [pltpu.VMEM((GRP, CHUNK), jnp.int32) for _ in range(4)]
        + [pltpu.VMEM((CHUNK, DIN), jnp.float32) for _ in range(NBUF)]
        + [pltpu.SemaphoreType.DMA for _ in range(NBUF + 6)]
    )

    def body(x_hbm, src_hbm, dst_hbm, s_hbm, acc, *rs):
        src_bs = rs[0:2]
        dst_bs = rs[2:4]
        rows = rs[4:4 + NBUF]
        sems = rs[4 + NBUF:4 + 2 * NBUF]
        isems = rs[4 + 2 * NBUF:6 + 2 * NBUF]
        wsems = rs[6 + 2 * NBUF:8 + 2 * NBUF]
        ssems = rs[8 + 2 * NBUF:10 + 2 * NBUF]
        c = lax.axis_index("c")
        s = lax.axis_index("s")
        wid = s * nc + c
        row0 = s * rpt
        out0 = c * N_PAD + row0

        z16 = jnp.zeros((16,), jnp.float32)

        def fill_zero(i, carry):
            for j in range(DIN // 16):
                rows[0][i, pl.ds(j * 16, 16)] = z16
            return carry

        lax.fori_loop(0, CHUNK, fill_zero, 0)

        for kk in range(nrc):
            pltpu.sync_copy(rows[0], acc.at[pl.ds(row0 + kk * CHUNK, CHUNK)])

        if not with_gather:
            o16 = jnp.ones((16,), jnp.float32)

            def fill_one(i, carry):
                for j in range(DIN // 16):
                    rows[0][i, pl.ds(j * 16, 16)] = o16
                return carry

            lax.fori_loop(0, CHUNK, fill_one, 0)

        plsc.subcore_barrier()

        g0 = wid * cpw

        def idx_fire(g, bi):
            gbase = g0 + g * GRP
            pltpu.async_copy(dst_hbm.at[pl.ds(gbase, GRP)], dst_bs[bi],
                             isems[bi])
            if with_gather:
                pltpu.async_copy(src_hbm.at[pl.ds(gbase, GRP)], src_bs[bi],
                                 isems[bi])

        def idx_wait(g, bi):
            gbase = g0 + g * GRP
            pltpu.make_async_copy(dst_hbm.at[pl.ds(gbase, GRP)], dst_bs[bi],
                                  isems[bi]).wait()
            if with_gather:
                pltpu.make_async_copy(src_hbm.at[pl.ds(gbase, GRP)],
                                      src_bs[bi], isems[bi]).wait()

        def process_group(g, bi):
            src_b = src_bs[bi]
            dst_b = dst_bs[bi]
            idx_wait(g, bi)
            idx_fire(g + 1, 1 - bi)
            if with_gather:
                pltpu.async_copy(x_hbm.at[src_b.at[0]], rows[0], sems[0])
                for j in range(GRP):
                    b = j % 2
                    pltpu.make_async_copy(x_hbm.at[src_b.at[j]], rows[b],
                                          sems[b]).wait()
                    pltpu.async_copy(rows[b], acc.at[dst_b.at[j]], ssems[b],
                                     add=True)
                    if j + 1 < GRP:
                        if j >= 1:
                            pltpu.make_async_copy(
                                rows[1 - b], acc.at[dst_b.at[j - 1]],
                                ssems[1 - b]).wait()
                        pltpu.async_copy(x_hbm.at[src_b.at[j + 1]],
                                         rows[1 - b], sems[1 - b])
                for j in (GRP - 2, GRP - 1):
                    b = j % 2
                    pltpu.make_async_copy(rows[b], acc.at[dst_b.at[j]],
                                          ssems[b]).wait()
            else:
                for j in range(GRP):
                    pltpu.async_copy(rows[0], acc.at[dst_b.at[j]],
                                     ssems[0], add=True)
                for j in range(GRP):
                    pltpu.make_async_copy(rows[0], acc.at[dst_b.at[j]],
                                          ssems[0]).wait()

        idx_fire(0, 0)

        def group_body(t, carry):
            g = 2 * t
            process_group(g, 0)
            process_group(g + 1, 1)
            return carry

        assert gpw % 2 == 0
        lax.fori_loop(0, gpw // 2, group_body, 0)
        idx_wait(gpw, 0)

        plsc.subcore_barrier()

        for kk in range(nrc):
            b = kk % 2
            if kk >= 2:
                pltpu.make_async_copy(
                    rows[b],
                    s_hbm.at[pl.ds(out0 + (kk - 2) * CHUNK, CHUNK)],
                    wsems[b]).wait()
            pltpu.sync_copy(acc.at[pl.ds(row0 + kk * CHUNK, CHUNK)], rows[b])
            pltpu.async_copy(rows[b],
                             s_hbm.at[pl.ds(out0 + kk * CHUNK, CHUNK)],
                             wsems[b])
        for kk in range(max(0, nrc - 2), nrc):
            b = kk % 2
            pltpu.make_async_copy(
                rows[b], s_hbm.at[pl.ds(out0 + kk * CHUNK, CHUNK)],
                wsems[b]).wait()

    return pl.kernel(body, out_type=out_type, mesh=mesh,
                     scratch_types=scratch)


@functools.lru_cache(maxsize=None)
def _make_tc_layer(relu: bool, final: bool, nc: int, bn: int = 640):
    grid = (pl.cdiv(N, bn),)
    blk = N_PAD // bn

    def body(*refs):
        s_refs = refs[0:nc]
        d_refs = refs[nc:2 * nc]
        if final:
            h_ref, wl_ref, wr_ref, b_ref, wf_ref, bf_ref, out_ref = refs[2 * nc:]
        else:
            h_ref, wl_ref, wr_ref, b_ref, out_ref = refs[2 * nc:]
        ssum = s_refs[0][...]
        dsum = d_refs[0][:, :1]
        for i in range(1, nc):
            ssum = ssum + s_refs[i][...]
            dsum = dsum + d_refs[i][:, :1]
        agg = ssum * (1.0 / jnp.maximum(dsum, 1.0))
        dn = (((1,), (1,)), ((), ()))
        out = (lax.dot_general(agg, wl_ref[...], dn,
                               precision=lax.Precision.HIGHEST,
                               preferred_element_type=jnp.float32)
               + lax.dot_general(h_ref[...], wr_ref[...], dn,
                                 precision=lax.Precision.HIGHEST,
                                 preferred_element_type=jnp.float32)
               + b_ref[...])
        if relu:
            out = jnp.maximum(out, 0.0)
        if final:
            out = (lax.dot_general(out, wf_ref[...], dn,
                                   precision=lax.Precision.HIGHEST,
                                   preferred_element_type=jnp.float32)
                   + bf_ref[...])
        out_ref[...] = out

    dout = DOUT if final else DIN

    def part_spec(core):
        return pl.BlockSpec((bn, DIN),
                            lambda i, core=core: (core * blk + i, 0))

    in_specs = ([part_spec(c) for c in range(nc)]
                + [part_spec(c) for c in range(nc)]
                + [
        pl.BlockSpec((bn, DIN), lambda i: (i, 0)),
        pl.BlockSpec((DIN, DIN), lambda i: (0, 0)),
        pl.BlockSpec((DIN, DIN), lambda i: (0, 0)),
        pl.BlockSpec((1, DIN), lambda i: (0, 0)),
    ])
    if final:
        in_specs += [
            pl.BlockSpec((DOUT, DIN), lambda i: (0, 0)),
            pl.BlockSpec((1, DOUT), lambda i: (0, 0)),
        ]

    def call(s_flat, deg_flat, h, *weights):
        args = ([s_flat] * nc) + ([deg_flat] * nc) + [h] + list(weights)
        return pl.pallas_call(
            body,
            grid=grid,
            in_specs=in_specs,
            out_specs=pl.BlockSpec((bn, dout), lambda i: (i, 0)),
            out_shape=jax.ShapeDtypeStruct((N, dout), jnp.float32),
        )(*args)

    return call


def kernel(x, edge_index, W1_l, W1_r, b1, W2_l, W2_r, b2, W3_l, W3_r, b3,
           W_lin, b_lin):
    nc, ns = _sc_dims()
    nw = nc * ns
    cpw = 2 * GRP * pl.cdiv(E, nw * CHUNK * 2 * GRP)
    e_pad = (nw * cpw + GRP) * CHUNK - E

    src = edge_index[0]
    dst = edge_index[1]
    if e_pad:
        pad_iota = jnp.arange(e_pad, dtype=jnp.int32)
        src = jnp.concatenate([src, pad_iota % N])
        dst = jnp.concatenate([dst, N + pad_iota % (N_PAD - N)])
    src = src.reshape(nw * cpw + GRP, CHUNK)
    dst = dst.reshape(nw * cpw + GRP, CHUNK)

    agg = _make_sc_agg(True, nc, ns, cpw)
    deg_k = _make_sc_agg(False, nc, ns, cpw)
    tc_mid = _make_tc_layer(True, False, nc)
    tc_fin = _make_tc_layer(False, True, nc)

    deg = deg_k(x, src, dst)
    s1 = agg(x, src, dst)
    h1 = tc_mid(s1, deg, x, W1_l, W1_r, b1.reshape(1, DIN))
    s2 = agg(h1, src, dst)
    h2 = tc_mid(s2, deg, h1, W2_l, W2_r, b2.reshape(1, DIN))
    s3 = agg(h2, src, dst)
    out = tc_fin(s3, deg, h2, W3_l, W3_r, b3.reshape(1, DIN),
                 W_lin, b_lin.reshape(1, DOUT))
    return out


# speedup vs baseline: 1.0894x; 1.0894x over previous
import functools

import jax
import jax.numpy as jnp
from jax import lax
from jax.experimental import pallas as pl
from jax.experimental.pallas import tpu as pltpu
from jax.experimental.pallas import tpu_sc as plsc

N = 10000
N_PAD = 10240
DIN = 128
DOUT = 64
E = 320000
CHUNK = 128


def _sc_dims():
    info = plsc.get_sparse_core_info()
    return info.num_cores, info.num_subcores


GRP = 8
NBUF = 2


@functools.lru_cache(maxsize=None)
def _make_sc_agg(with_gather: bool, nc: int, ns: int, cpw: int):
    rpt = N_PAD // ns
    nrc = rpt // CHUNK
    gpw = cpw // GRP

    mesh = plsc.VectorSubcoreMesh(core_axis_name="c", subcore_axis_name="s")
    out_type = jax.ShapeDtypeStruct((nc * N_PAD, DIN), jnp.float32)

    scratch = (
        [pltpu.VMEM_SHARED((N_PAD, DIN), jnp.float32)]
        + [pltpu.VMEM((GRP, CHUNK), jnp.int32) for _ in range(4)]
        + [pltpu.VMEM((CHUNK, DIN), jnp.float32) for _ in range(NBUF)]
        + [pltpu.SemaphoreType.DMA for _ in range(NBUF + 6)]
    )

    def body(x_hbm, src_hbm, dst_hbm, s_hbm, acc, *rs):
        src_bs = rs[0:2]
        dst_bs = rs[2:4]
        rows = rs[4:4 + NBUF]
        sems = rs[4 + NBUF:4 + 2 * NBUF]
        isems = rs[4 + 2 * NBUF:6 + 2 * NBUF]
        wsems = rs[6 + 2 * NBUF:8 + 2 * NBUF]
        ssems = rs[8 + 2 * NBUF:10 + 2 * NBUF]
        c = lax.axis_index("c")
        s = lax.axis_index("s")
        wid = s * nc + c
        row0 = s * rpt
        out0 = c * N_PAD + row0

        z16 = jnp.zeros((16,), jnp.float32)

        def fill_zero(i, carry):
            for j in range(DIN // 16):
                rows[0][i, pl.ds(j * 16, 16)] = z16
            return carry

        lax.fori_loop(0, CHUNK, fill_zero, 0)

        for kk in range(nrc):
            pltpu.sync_copy(rows[0], acc.at[pl.ds(row0 + kk * CHUNK, CHUNK)])

        if not with_gather:
            o16 = jnp.ones((16,), jnp.float32)

            def fill_one(i, carry):
                for j in range(DIN // 16):
                    rows[0][i, pl.ds(j * 16, 16)] = o16
                return carry

            lax.fori_loop(0, CHUNK, fill_one, 0)

        plsc.subcore_barrier()

        g0 = wid * cpw

        def idx_fire(g, bi):
            gbase = g0 + g * GRP
            pltpu.async_copy(dst_hbm.at[pl.ds(gbase, GRP)], dst_bs[bi],
                             isems[bi])
            if with_gather:
                pltpu.async_copy(src_hbm.at[pl.ds(gbase, GRP)], src_bs[bi],
                                 isems[bi])

        def idx_wait(g, bi):
            gbase = g0 + g * GRP
            pltpu.make_async_copy(dst_hbm.at[pl.ds(gbase, GRP)], dst_bs[bi],
                                  isems[bi]).wait()
            if with_gather:
                pltpu.make_async_copy(src_hbm.at[pl.ds(gbase, GRP)],
                                      src_bs[bi], isems[bi]).wait()

        def process_group(g, bi):
            src_b = src_bs[bi]
            dst_b = dst_bs[bi]
            idx_wait(g, bi)
            idx_fire(g + 1, 1 - bi)
            if with_gather:
                for j in range(min(NBUF, GRP)):
                    pltpu.async_copy(x_hbm.at[src_b.at[j]], rows[j % NBUF],
                                     sems[j % NBUF])
                for j in range(GRP):
                    b = j % NBUF
                    pltpu.make_async_copy(x_hbm.at[src_b.at[j]], rows[b],
                                          sems[b]).wait()
                    pltpu.sync_copy(rows[b], acc.at[dst_b.at[j]], add=True)
                    if j + NBUF < GRP:
                        pltpu.async_copy(x_hbm.at[src_b.at[j + NBUF]],
                                         rows[b], sems[b])
            else:
                for j in range(GRP):
                    pltpu.async_copy(rows[0], acc.at[dst_b.at[j]],
                                     ssems[0], add=True)
                for j in range(GRP):
                    pltpu.make_async_copy(rows[0], acc.at[dst_b.at[j]],
                                          ssems[0]).wait()

        idx_fire(0, 0)

        def group_body(t, carry):
            g = 2 * t
            process_group(g, 0)
            process_group(g + 1, 1)
            return carry

        assert gpw % 2 == 0
        lax.fori_loop(0, gpw // 2, group_body, 0)
        idx_wait(gpw, 0)

        plsc.subcore_barrier()

        for kk in range(nrc):
            b = kk % 2
            if kk >= 2:
                pltpu.make_async_copy(
                    rows[b],
                    s_hbm.at[pl.ds(out0 + (kk - 2) * CHUNK, CHUNK)],
                    wsems[b]).wait()
            pltpu.sync_copy(acc.at[pl.ds(row0 + kk * CHUNK, CHUNK)], rows[b])
            pltpu.async_copy(rows[b],
                             s_hbm.at[pl.ds(out0 + kk * CHUNK, CHUNK)],
                             wsems[b])
        for kk in range(max(0, nrc - 2), nrc):
            b = kk % 2
            pltpu.make_async_copy(
                rows[b], s_hbm.at[pl.ds(out0 + kk * CHUNK, CHUNK)],
                wsems[b]).wait()

    return pl.kernel(body, out_type=out_type, mesh=mesh,
                     scratch_types=scratch)


@functools.lru_cache(maxsize=None)
def _make_tc_layer(relu: bool, final: bool, nc: int, bn: int = 640):
    grid = (pl.cdiv(N, bn),)
    blk = N_PAD // bn

    def body(*refs):
        s_refs = refs[0:nc]
        d_refs = refs[nc:2 * nc]
        if final:
            h_ref, wl_ref, wr_ref, b_ref, wf_ref, bf_ref, out_ref = refs[2 * nc:]
        else:
            h_ref, wl_ref, wr_ref, b_ref, out_ref = refs[2 * nc:]
        ssum = s_refs[0][...]
        dsum = d_refs[0][:, :1]
        for i in range(1, nc):
            ssum = ssum + s_refs[i][...]
            dsum = dsum + d_refs[i][:, :1]
        agg = ssum * (1.0 / jnp.maximum(dsum, 1.0))
        dn = (((1,), (1,)), ((), ()))
        out = (lax.dot_general(agg, wl_ref[...], dn,
                               precision=lax.Precision.HIGHEST,
                               preferred_element_type=jnp.float32)
               + lax.dot_general(h_ref[...], wr_ref[...], dn,
                                 precision=lax.Precision.HIGHEST,
                                 preferred_element_type=jnp.float32)
               + b_ref[...])
        if relu:
            out = jnp.maximum(out, 0.0)
        if final:
            out = (lax.dot_general(out, wf_ref[...], dn,
                                   precision=lax.Precision.HIGHEST,
                                   preferred_element_type=jnp.float32)
                   + bf_ref[...])
        out_ref[...] = out

    dout = DOUT if final else DIN

    def part_spec(core):
        return pl.BlockSpec((bn, DIN),
                            lambda i, core=core: (core * blk + i, 0))

    in_specs = ([part_spec(c) for c in range(nc)]
                + [part_spec(c) for c in range(nc)]
                + [
        pl.BlockSpec((bn, DIN), lambda i: (i, 0)),
        pl.BlockSpec((DIN, DIN), lambda i: (0, 0)),
        pl.BlockSpec((DIN, DIN), lambda i: (0, 0)),
        pl.BlockSpec((1, DIN), lambda i: (0, 0)),
    ])
    if final:
        in_specs += [
            pl.BlockSpec((DOUT, DIN), lambda i: (0, 0)),
            pl.BlockSpec((1, DOUT), lambda i: (0, 0)),
        ]

    def call(s_flat, deg_flat, h, *weights):
        args = ([s_flat] * nc) + ([deg_flat] * nc) + [h] + list(weights)
        return pl.pallas_call(
            body,
            grid=grid,
            in_specs=in_specs,
            out_specs=pl.BlockSpec((bn, dout), lambda i: (i, 0)),
            out_shape=jax.ShapeDtypeStruct((N, dout), jnp.float32),
        )(*args)

    return call


def kernel(x, edge_index, W1_l, W1_r, b1, W2_l, W2_r, b2, W3_l, W3_r, b3,
           W_lin, b_lin):
    nc, ns = _sc_dims()
    nw = nc * ns
    cpw = 2 * GRP * pl.cdiv(E, nw * CHUNK * 2 * GRP)
    e_pad = (nw * cpw + GRP) * CHUNK - E

    src = edge_index[0]
    dst = edge_index[1]
    if e_pad:
        pad_iota = jnp.arange(e_pad, dtype=jnp.int32)
        src = jnp.concatenate([src, pad_iota % N])
        dst = jnp.concatenate([dst, N + pad_iota % (N_PAD - N)])
    src = src.reshape(nw * cpw + GRP, CHUNK)
    dst = dst.reshape(nw * cpw + GRP, CHUNK)

    agg = _make_sc_agg(True, nc, ns, cpw)
    deg_k = _make_sc_agg(False, nc, ns, cpw)
    tc_mid = _make_tc_layer(True, False, nc)
    tc_fin = _make_tc_layer(False, True, nc)

    deg = deg_k(x, src, dst)
    s1 = agg(x, src, dst)
    h1 = tc_mid(s1, deg, x, W1_l, W1_r, b1.reshape(1, DIN))
    s2 = agg(h1, src, dst)
    h2 = tc_mid(s2, deg, h1, W2_l, W2_r, b2.reshape(1, DIN))
    s3 = agg(h2, src, dst)
    out = tc_fin(s3, deg, h2, W3_l, W3_r, b3.reshape(1, DIN),
                 W_lin, b_lin.reshape(1, DOUT))
    return out


# speedup vs baseline: 1.1020x; 1.0115x over previous
import functools

import jax
import jax.numpy as jnp
from jax import lax
from jax.experimental import pallas as pl
from jax.experimental.pallas import tpu as pltpu
from jax.experimental.pallas import tpu_sc as plsc

N = 10000
N_PAD = 10240
DIN = 128
DOUT = 64
E = 320000
CHUNK = 128


def _sc_dims():
    info = plsc.get_sparse_core_info()
    return info.num_cores, info.num_subcores


GRP = 8
NBUF = 2


@functools.lru_cache(maxsize=None)
def _make_sc_agg(fuse_deg: bool, nc: int, ns: int, cpw: int):
    rpt = N_PAD // ns
    nrc = rpt // CHUNK
    gpw = cpw // GRP
    assert gpw % 2 == 0

    mesh = plsc.VectorSubcoreMesh(core_axis_name="c", subcore_axis_name="s")
    st = jax.ShapeDtypeStruct((nc * N_PAD, DIN), jnp.float32)
    out_type = (st, st) if fuse_deg else st

    scratch = (
        [pltpu.VMEM_SHARED((N_PAD, DIN), jnp.float32)]
        + [pltpu.VMEM((GRP, CHUNK), jnp.int32) for _ in range(4)]
        + [pltpu.VMEM((CHUNK, DIN), jnp.float32) for _ in range(NBUF)]
        + [pltpu.SemaphoreType.DMA for _ in range(NBUF + 6)]
    )

    def body(x_hbm, src_hbm, dst_hbm, *rest):
        if fuse_deg:
            s_hbm, deg_hbm = rest[0:2]
            rs = rest[2:]
        else:
            s_hbm = rest[0]
            rs = rest[1:]
        acc = rs[0]
        src_bs = rs[1:3]
        dst_bs = rs[3:5]
        rows = rs[5:5 + NBUF]
        sems = rs[5 + NBUF:5 + 2 * NBUF]
        isems = rs[5 + 2 * NBUF:7 + 2 * NBUF]
        wsems = rs[7 + 2 * NBUF:9 + 2 * NBUF]
        ssems = rs[9 + 2 * NBUF:11 + 2 * NBUF]
        c = lax.axis_index("c")
        s = lax.axis_index("s")
        wid = s * nc + c
        row0 = s * rpt
        out0 = c * N_PAD + row0
        g0 = wid * cpw

        def fill(row_ref, vec):
            def f(i, carry):
                for j in range(DIN // 16):
                    row_ref[i, pl.ds(j * 16, 16)] = vec
                return carry
            lax.fori_loop(0, CHUNK, f, 0)

        def zero_slab():
            for kk in range(nrc):
                pltpu.sync_copy(rows[0],
                                acc.at[pl.ds(row0 + kk * CHUNK, CHUNK)])

        def idx_fire(g, bi, use_gather):
            gbase = g0 + g * GRP
            pltpu.async_copy(dst_hbm.at[pl.ds(gbase, GRP)], dst_bs[bi],
                             isems[bi])
            if use_gather:
                pltpu.async_copy(src_hbm.at[pl.ds(gbase, GRP)], src_bs[bi],
                                 isems[bi])

        def idx_wait(g, bi, use_gather):
            gbase = g0 + g * GRP
            pltpu.make_async_copy(dst_hbm.at[pl.ds(gbase, GRP)], dst_bs[bi],
                                  isems[bi]).wait()
            if use_gather:
                pltpu.make_async_copy(src_hbm.at[pl.ds(gbase, GRP)],
                                      src_bs[bi], isems[bi]).wait()

        def edge_pass(use_gather):
            def process_group(g, bi):
                src_b = src_bs[bi]
                dst_b = dst_bs[bi]
                idx_wait(g, bi, use_gather)
                idx_fire(g + 1, 1 - bi, use_gather)
                if use_gather:
                    for j in range(min(NBUF, GRP)):
                        pltpu.async_copy(x_hbm.at[src_b.at[j]],
                                         rows[j % NBUF], sems[j % NBUF])
                    for j in range(GRP):
                        b = j % NBUF
                        pltpu.make_async_copy(x_hbm.at[src_b.at[j]], rows[b],
                                              sems[b]).wait()
                        pltpu.sync_copy(rows[b], acc.at[dst_b.at[j]],
                                        add=True)
                        if j + NBUF < GRP:
                            pltpu.async_copy(x_hbm.at[src_b.at[j + NBUF]],
                                             rows[b], sems[b])
                else:
                    for j in range(GRP):
                        pltpu.async_copy(rows[1], acc.at[dst_b.at[j]],
                                         ssems[0], add=True)
                    for j in range(GRP):
                        pltpu.make_async_copy(rows[1], acc.at[dst_b.at[j]],
                                              ssems[0]).wait()

            idx_fire(0, 0, use_gather)

            def group_body(t, carry):
                process_group(2 * t, 0)
                process_group(2 * t + 1, 1)
                return carry

            lax.fori_loop(0, gpw // 2, group_body, 0)
            idx_wait(gpw, 0, use_gather)

        def copyout(dst_ref):
            for kk in range(nrc):
                b = kk % 2
                if kk >= 2:
                    pltpu.make_async_copy(
                        rows[b],
                        dst_ref.at[pl.ds(out0 + (kk - 2) * CHUNK, CHUNK)],
                        wsems[b]).wait()
                pltpu.sync_copy(acc.at[pl.ds(row0 + kk * CHUNK, CHUNK)],
                                rows[b])
                pltpu.async_copy(rows[b],
                                 dst_ref.at[pl.ds(out0 + kk * CHUNK, CHUNK)],
                                 wsems[b])
            for kk in range(max(0, nrc - 2), nrc):
                b = kk % 2
                pltpu.make_async_copy(
                    rows[b], dst_ref.at[pl.ds(out0 + kk * CHUNK, CHUNK)],
                    wsems[b]).wait()

        z16 = jnp.zeros((16,), jnp.float32)
        fill(rows[0], z16)
        zero_slab()
        plsc.subcore_barrier()
        edge_pass(True)
        plsc.subcore_barrier()
        copyout(s_hbm)

        if fuse_deg:
            fill(rows[0], z16)
            fill(rows[1], jnp.ones((16,), jnp.float32))
            zero_slab()
            plsc.subcore_barrier()
            edge_pass(False)
            plsc.subcore_barrier()
            copyout(deg_hbm)

    return pl.kernel(body, out_type=out_type, mesh=mesh,
                     scratch_types=scratch)


@functools.lru_cache(maxsize=None)
def _make_tc_layer(relu: bool, final: bool, nc: int, bn: int = 640):
    grid = (pl.cdiv(N, bn),)
    blk = N_PAD // bn

    def body(*refs):
        s_refs = refs[0:nc]
        d_refs = refs[nc:2 * nc]
        if final:
            h_ref, wl_ref, wr_ref, b_ref, wf_ref, bf_ref, out_ref = refs[2 * nc:]
        else:
            h_ref, wl_ref, wr_ref, b_ref, out_ref = refs[2 * nc:]
        ssum = s_refs[0][...]
        dsum = d_refs[0][:, :1]
        for i in range(1, nc):
            ssum = ssum + s_refs[i][...]
            dsum = dsum + d_refs[i][:, :1]
        agg = ssum * (1.0 / jnp.maximum(dsum, 1.0))
        dn = (((1,), (1,)), ((), ()))
        out = (lax.dot_general(agg, wl_ref[...], dn,
                               precision=lax.Precision.HIGHEST,
                               preferred_element_type=jnp.float32)
               + lax.dot_general(h_ref[...], wr_ref[...], dn,
                                 precision=lax.Precision.HIGHEST,
                                 preferred_element_type=jnp.float32)
               + b_ref[...])
        if relu:
            out = jnp.maximum(out, 0.0)
        if final:
            out = (lax.dot_general(out, wf_ref[...], dn,
                                   precision=lax.Precision.HIGHEST,
                                   preferred_element_type=jnp.float32)
                   + bf_ref[...])
        out_ref[...] = out

    dout = DOUT if final else DIN

    def part_spec(core):
        return pl.BlockSpec((bn, DIN),
                            lambda i, core=core: (core * blk + i, 0))

    in_specs = ([part_spec(c) for c in range(nc)]
                + [part_spec(c) for c in range(nc)]
                + [
        pl.BlockSpec((bn, DIN), lambda i: (i, 0)),
        pl.BlockSpec((DIN, DIN), lambda i: (0, 0)),
        pl.BlockSpec((DIN, DIN), lambda i: (0, 0)),
        pl.BlockSpec((1, DIN), lambda i: (0, 0)),
    ])
    if final:
        in_specs += [
            pl.BlockSpec((DOUT, DIN), lambda i: (0, 0)),
            pl.BlockSpec((1, DOUT), lambda i: (0, 0)),
        ]

    def call(s_flat, deg_flat, h, *weights):
        args = ([s_flat] * nc) + ([deg_flat] * nc) + [h] + list(weights)
        return pl.pallas_call(
            body,
            grid=grid,
            in_specs=in_specs,
            out_specs=pl.BlockSpec((bn, dout), lambda i: (i, 0)),
            out_shape=jax.ShapeDtypeStruct((N, dout), jnp.float32),
        )(*args)

    return call


def kernel(x, edge_index, W1_l, W1_r, b1, W2_l, W2_r, b2, W3_l, W3_r, b3,
           W_lin, b_lin):
    nc, ns = _sc_dims()
    nw = nc * ns
    cpw = 2 * GRP * pl.cdiv(E, nw * CHUNK * 2 * GRP)
    e_pad = (nw * cpw + GRP) * CHUNK - E

    src = edge_index[0]
    dst = edge_index[1]
    if e_pad:
        pad_iota = jnp.arange(e_pad, dtype=jnp.int32)
        src = jnp.concatenate([src, pad_iota % N])
        dst = jnp.concatenate([dst, N + pad_iota % (N_PAD - N)])
    src = src.reshape(nw * cpw + GRP, CHUNK)
    dst = dst.reshape(nw * cpw + GRP, CHUNK)

    agg1 = _make_sc_agg(True, nc, ns, cpw)
    agg = _make_sc_agg(False, nc, ns, cpw)
    tc_mid = _make_tc_layer(True, False, nc)
    tc_fin = _make_tc_layer(False, True, nc)

    s1, deg = agg1(x, src, dst)
    h1 = tc_mid(s1, deg, x, W1_l, W1_r, b1.reshape(1, DIN))
    s2 = agg(h1, src, dst)
    h2 = tc_mid(s2, deg, h1, W2_l, W2_r, b2.reshape(1, DIN))
    s3 = agg(h2, src, dst)
    out = tc_fin(s3, deg, h2, W3_l, W3_r, b3.reshape(1, DIN),
                 W_lin, b_lin.reshape(1, DOUT))
    return out
